# baseline (speedup 1.0000x reference)
import math

import jax
import jax.numpy as jnp
from jax import lax
from jax.experimental import pallas as pl
from jax.experimental.pallas import tpu as pltpu
from jax.experimental.pallas import tpu_sc as plsc

D_MODEL = 64
SCALE = math.sqrt(D_MODEL)

NC = 2
NS = 16
NW = NC * NS

BATCH = 1024
SEQ = 200
ROWS_PER_W = BATCH // NW
NPAIR = ROWS_PER_W // 2


def _emb_kernel(table_hbm, x_hbm, out_hbm, idx_v, gb0, gb1, gs0, gs1, ps0, ps1):
    wid = lax.axis_index("s") * NC + lax.axis_index("c")
    row0 = wid * ROWS_PER_W

    pltpu.sync_copy(x_hbm.at[pl.ds(row0, ROWS_PER_W)], idx_v)

    def issue_gathers(r, gb, gs):
        def window(w):
            t0 = w * 16
            v = idx_v[r, pl.ds(t0, 16)]
            for j in range(16):
                pltpu.async_copy(
                    table_hbm.at[pl.ds(v[j], 1)], gb.at[pl.ds(t0 + j, 1)], gs)

        pl.loop(0, SEQ // 16)(window)
        vt = idx_v[r, pl.ds(SEQ - 16, 16)]
        for j in range(8):
            pltpu.async_copy(
                table_hbm.at[pl.ds(vt[8 + j], 1)],
                gb.at[pl.ds(SEQ - 8 + j, 1)], gs)

    def wait_gathers(gb, gs):
        pltpu.make_async_copy(table_hbm.at[pl.ds(0, SEQ)], gb, gs).wait()

    def start_put(r, gb, ps):
        pltpu.async_copy(gb, out_hbm.at[row0 + r], ps)

    def wait_put(gb, ps):
        pltpu.make_async_copy(gb, out_hbm.at[row0], ps).wait()

    issue_gathers(0, gb0, gs0)
    issue_gathers(1, gb1, gs1)
    wait_gathers(gb0, gs0)
    start_put(0, gb0, ps0)
    wait_gathers(gb1, gs1)
    start_put(1, gb1, ps1)

    def pair_body(g):
        r = 2 * g
        wait_put(gb0, ps0)
        issue_gathers(r, gb0, gs0)
        wait_gathers(gb0, gs0)
        start_put(r, gb0, ps0)
        wait_put(gb1, ps1)
        issue_gathers(r + 1, gb1, gs1)
        wait_gathers(gb1, gs1)
        start_put(r + 1, gb1, ps1)

    pl.loop(1, NPAIR)(pair_body)

    wait_put(gb0, ps0)
    wait_put(gb1, ps1)


@jax.jit
def kernel(x, table):
    mesh = plsc.VectorSubcoreMesh(core_axis_name="c", subcore_axis_name="s")
    run = pl.kernel(
        _emb_kernel,
        out_type=jax.ShapeDtypeStruct((BATCH, SEQ, D_MODEL), jnp.float32),
        mesh=mesh,
        scratch_types=(
            [pltpu.VMEM((ROWS_PER_W, SEQ), jnp.int32),
             pltpu.VMEM((SEQ, D_MODEL), jnp.float32),
             pltpu.VMEM((SEQ, D_MODEL), jnp.float32),
             pltpu.SemaphoreType.DMA,
             pltpu.SemaphoreType.DMA,
             pltpu.SemaphoreType.DMA,
             pltpu.SemaphoreType.DMA]
        ),
        compiler_params=pltpu.CompilerParams(use_tc_tiling_on_sc=True),
    )
    return run(table * SCALE, x.astype(jnp.int32))


# speedup vs baseline: 1.3493x; 1.3493x over previous
import math

import jax
import jax.numpy as jnp
from jax import lax
from jax.experimental import pallas as pl
from jax.experimental.pallas import tpu as pltpu
from jax.experimental.pallas import tpu_sc as plsc

D_MODEL = 64
SCALE = math.sqrt(D_MODEL)

NC = 2
NS = 16
NW = NC * NS

BATCH = 1024
SEQ = 200
ROWS_PER_W = BATCH // NW
NPAIR = ROWS_PER_W // 2


def _emb_kernel(table_hbm, x_hbm, out_hbm, idx_v, gb0, gb1, gs0, gs1, ps0, ps1):
    wid = lax.axis_index("s") * NC + lax.axis_index("c")
    row0 = wid * ROWS_PER_W

    pltpu.sync_copy(x_hbm.at[pl.ds(row0, ROWS_PER_W)], idx_v)

    def issue_gathers(r, gb, gs):
        def window(w):
            t0 = w * 16
            v = idx_v[r, pl.ds(t0, 16)]
            for j in range(16):
                pltpu.async_copy(
                    table_hbm.at[pl.ds(v[j], 1)], gb.at[pl.ds(t0 + j, 1)], gs)

        pl.loop(0, SEQ // 16)(window)
        vt = idx_v[r, pl.ds(SEQ - 16, 16)]
        for j in range(8):
            pltpu.async_copy(
                table_hbm.at[pl.ds(vt[8 + j], 1)],
                gb.at[pl.ds(SEQ - 8 + j, 1)], gs)

    def wait_gathers(gb, gs):
        pltpu.make_async_copy(table_hbm.at[pl.ds(0, SEQ)], gb, gs).wait()

    def mul_row(gb):
        def body(t):
            for d in range(4):
                sl = pl.ds(d * 16, 16)
                gb[t, sl] = gb[t, sl] * SCALE

        pl.loop(0, SEQ, unroll=4)(body)

    def start_put(r, gb, ps):
        pltpu.async_copy(gb, out_hbm.at[row0 + r], ps)

    def wait_put(gb, ps):
        pltpu.make_async_copy(gb, out_hbm.at[row0], ps).wait()

    issue_gathers(0, gb0, gs0)
    issue_gathers(1, gb1, gs1)
    wait_gathers(gb0, gs0)
    mul_row(gb0)
    start_put(0, gb0, ps0)
    wait_gathers(gb1, gs1)
    mul_row(gb1)
    start_put(1, gb1, ps1)

    def pair_body(g):
        r = 2 * g
        wait_put(gb0, ps0)
        issue_gathers(r, gb0, gs0)
        wait_gathers(gb0, gs0)
        mul_row(gb0)
        start_put(r, gb0, ps0)
        wait_put(gb1, ps1)
        issue_gathers(r + 1, gb1, gs1)
        wait_gathers(gb1, gs1)
        mul_row(gb1)
        start_put(r + 1, gb1, ps1)

    pl.loop(1, NPAIR)(pair_body)

    wait_put(gb0, ps0)
    wait_put(gb1, ps1)


@jax.jit
def kernel(x, table):
    mesh = plsc.VectorSubcoreMesh(core_axis_name="c", subcore_axis_name="s")
    run = pl.kernel(
        _emb_kernel,
        out_type=jax.ShapeDtypeStruct((BATCH, SEQ, D_MODEL), jnp.float32),
        mesh=mesh,
        scratch_types=(
            [pltpu.VMEM((ROWS_PER_W, SEQ), jnp.int32),
             pltpu.VMEM((SEQ, D_MODEL), jnp.float32),
             pltpu.VMEM((SEQ, D_MODEL), jnp.float32),
             pltpu.SemaphoreType.DMA,
             pltpu.SemaphoreType.DMA,
             pltpu.SemaphoreType.DMA,
             pltpu.SemaphoreType.DMA]
        ),
        compiler_params=pltpu.CompilerParams(use_tc_tiling_on_sc=True),
    )
    return run(table, x.astype(jnp.int32))
